# baseline (speedup 1.0000x reference)
import jax
import jax.numpy as jnp
from jax.experimental import pallas as pl

_NUM_EXPERTS = 8
_HIDDEN = 1024
_INTER = 1024
_T = 64
_TOPK = 2
_IC = 512
_NC = _INTER // _IC


def _moe_kernel(sel_ref, x_ref, g_ref, u_ref, dn_ref, out_ref):
    e = pl.program_id(0)
    c = pl.program_id(1)

    @pl.when(jnp.logical_and(e == 0, c == 0))
    def _init():
        out_ref[...] = jnp.zeros_like(out_ref)

    x = x_ref[...]
    g = jax.lax.dot_general(
        x, g_ref[0], (((1,), (1,)), ((), ())),
        preferred_element_type=jnp.float32)
    u = jax.lax.dot_general(
        x, u_ref[0], (((1,), (1,)), ((), ())),
        preferred_element_type=jnp.float32)
    inter = g * jax.nn.sigmoid(g) * u
    o = jax.lax.dot_general(
        inter, dn_ref[0], (((1,), (1,)), ((), ())),
        preferred_element_type=jnp.float32)
    sel = sel_ref[...]
    for k in range(_TOPK):
        mk = (sel[:, k:k + 1] == e).astype(jnp.float32)
        out_ref[:, k * _HIDDEN:(k + 1) * _HIDDEN] += mk * o


def kernel(hidden_states, selected_experts, gate_up_proj, down_proj):
    out_flat = pl.pallas_call(
        _moe_kernel,
        grid=(_NUM_EXPERTS, _NC),
        in_specs=[
            pl.BlockSpec((_T, _TOPK), lambda e, c: (0, 0)),
            pl.BlockSpec((_T, _HIDDEN), lambda e, c: (0, 0)),
            pl.BlockSpec((1, _IC, _HIDDEN), lambda e, c: (e, c, 0)),
            pl.BlockSpec((1, _IC, _HIDDEN), lambda e, c: (e, c + _NC, 0)),
            pl.BlockSpec((1, _HIDDEN, _IC), lambda e, c: (e, 0, c)),
        ],
        out_specs=pl.BlockSpec((_T, _TOPK * _HIDDEN), lambda e, c: (0, 0)),
        out_shape=jax.ShapeDtypeStruct((_T, _TOPK * _HIDDEN), jnp.float32),
    )(selected_experts, hidden_states, gate_up_proj, gate_up_proj, down_proj)
    return out_flat.reshape(_T, _TOPK, _HIDDEN)


# speedup vs baseline: 1.0604x; 1.0604x over previous
import jax
import jax.numpy as jnp
from jax.experimental import pallas as pl

_NUM_EXPERTS = 8
_HIDDEN = 1024
_INTER = 1024
_T = 64
_TOPK = 2


def _moe_kernel(sel_ref, x_ref, g_ref, u_ref, dn_ref, out_ref):
    e = pl.program_id(0)
    x = x_ref[...]
    g = jax.lax.dot_general(
        x, g_ref[0], (((1,), (1,)), ((), ())),
        preferred_element_type=jnp.float32)
    u = jax.lax.dot_general(
        x, u_ref[0], (((1,), (1,)), ((), ())),
        preferred_element_type=jnp.float32)
    inter = g * jax.nn.sigmoid(g) * u
    o = jax.lax.dot_general(
        inter, dn_ref[0], (((1,), (1,)), ((), ())),
        preferred_element_type=jnp.float32)
    sel = sel_ref[...]
    for k in range(_TOPK):
        mk = sel[:, k:k + 1] == e
        cur = out_ref[:, k * _HIDDEN:(k + 1) * _HIDDEN]
        out_ref[:, k * _HIDDEN:(k + 1) * _HIDDEN] = jnp.where(mk, o, cur)


def kernel(hidden_states, selected_experts, gate_up_proj, down_proj):
    out_flat = pl.pallas_call(
        _moe_kernel,
        grid=(_NUM_EXPERTS,),
        in_specs=[
            pl.BlockSpec((_T, _TOPK), lambda e: (0, 0)),
            pl.BlockSpec((_T, _HIDDEN), lambda e: (0, 0)),
            pl.BlockSpec((1, _INTER, _HIDDEN), lambda e: (e, 0, 0)),
            pl.BlockSpec((1, _INTER, _HIDDEN), lambda e: (e, 1, 0)),
            pl.BlockSpec((1, _HIDDEN, _INTER), lambda e: (e, 0, 0)),
        ],
        out_specs=pl.BlockSpec((_T, _TOPK * _HIDDEN), lambda e: (0, 0)),
        out_shape=jax.ShapeDtypeStruct((_T, _TOPK * _HIDDEN), jnp.float32),
    )(selected_experts, hidden_states, gate_up_proj, gate_up_proj, down_proj)
    return out_flat.reshape(_T, _TOPK, _HIDDEN)
